# Initial kernel scaffold; baseline (speedup 1.0000x reference)
import jax
import jax.numpy as jnp
from jax.experimental import pallas as pl


def kernel(x, table):
    raise NotImplementedError("write your pallas kernel here")


# speedup vs baseline: 1.0899x; 1.0899x over previous
import functools

import jax
import jax.numpy as jnp
from jax import lax
from jax.experimental import pallas as pl
from jax.experimental.pallas import tpu as pltpu
from jax.experimental.pallas import tpu_sc as plsc

NC = 2
NS = 16
NW = NC * NS
CHUNK = 392


@functools.cache
def _build(n_chunks: int, d: int, dtype):
    b_per_w = n_chunks * CHUNK
    b_pad = b_per_w * NW
    mesh = plsc.VectorSubcoreMesh(core_axis_name="c", subcore_axis_name="s")

    @functools.partial(
        pl.kernel,
        mesh=mesh,
        out_type=jax.ShapeDtypeStruct((b_pad, d), dtype),
        scratch_types=[
            pltpu.VMEM((n_chunks, CHUNK), jnp.int32),
            pltpu.VMEM((2, CHUNK, d), dtype),
            pltpu.SemaphoreType.DMA,
            pltpu.SemaphoreType.DMA,
            pltpu.SemaphoreType.DMA,
            pltpu.SemaphoreType.DMA,
        ],
        compiler_params=pltpu.CompilerParams(use_tc_tiling_on_sc=False),
    )
    def gather_kernel(idx_hbm, table_hbm, out_hbm, idx_v, buf_v, g0, g1, s0, s1):
        wid = lax.axis_index("s") * NC + lax.axis_index("c")
        base = wid * b_per_w
        pltpu.sync_copy(idx_hbm.at[wid], idx_v)

        gsems = (g0, g1)
        ssems = (s0, s1)

        def gather_start(c):
            return pltpu.async_copy(
                table_hbm.at[idx_v.at[c]], buf_v.at[c % 2], gsems[c % 2]
            )

        def store_start(c):
            return pltpu.async_copy(
                buf_v.at[c % 2], out_hbm.at[pl.ds(base + c * CHUNK, CHUNK)],
                ssems[c % 2],
            )

        gh = [None] * n_chunks
        sh = [None] * n_chunks
        gh[0] = gather_start(0)
        for c in range(n_chunks):
            if c + 1 < n_chunks:
                if c >= 1:
                    sh[c - 1].wait()
                gh[c + 1] = gather_start(c + 1)
            gh[c].wait()
            sh[c] = store_start(c)
        if n_chunks >= 2:
            sh[n_chunks - 2].wait()
        sh[n_chunks - 1].wait()

    return gather_kernel


def kernel(x, table):
    n = x.shape[0]
    d = table.shape[1]
    per_round = NW * CHUNK
    n_chunks = -(-n // per_round)
    b_pad = n_chunks * per_round
    xi = x.astype(jnp.int32)
    if b_pad != n:
        xi = jnp.concatenate([xi, jnp.zeros((b_pad - n,), jnp.int32)])
    xi = xi.reshape(NW, n_chunks, CHUNK)
    out = _build(n_chunks, d, table.dtype)(xi, table)
    return out[:n]


# speedup vs baseline: 2.0326x; 1.8650x over previous
import functools

import jax
import jax.numpy as jnp
from jax import lax
from jax.experimental import pallas as pl
from jax.experimental.pallas import tpu as pltpu
from jax.experimental.pallas import tpu_sc as plsc

NC = 2
NS = 16
NW = NC * NS
NBUF = 4


def _chunk_rows(rows_per_worker: int) -> int:
    best = 1
    for c in range(1, 257):
        if rows_per_worker % c == 0:
            best = c
    return best


@functools.cache
def _build(rows_per_worker: int, chunk: int, d: int, dtype):
    n_chunks = rows_per_worker // chunk
    nbuf = min(NBUF, n_chunks)
    mesh = plsc.VectorSubcoreMesh(core_axis_name="c", subcore_axis_name="s")

    @functools.partial(
        pl.kernel,
        mesh=mesh,
        out_type=jax.ShapeDtypeStruct((NW * rows_per_worker, d), dtype),
        scratch_types=[
            pltpu.VMEM((n_chunks, chunk), jnp.int32),
            pltpu.VMEM((nbuf, chunk, d), dtype),
            [pltpu.SemaphoreType.DMA] * nbuf,
            [pltpu.SemaphoreType.DMA] * nbuf,
        ],
        compiler_params=pltpu.CompilerParams(use_tc_tiling_on_sc=False),
    )
    def gather_kernel(idx_hbm, table_hbm, out_hbm, idx_v, buf_v, gsems, ssems):
        wid = lax.axis_index("s") * NC + lax.axis_index("c")
        base = wid * rows_per_worker
        pltpu.sync_copy(idx_hbm.at[wid], idx_v)

        def gather_start(c):
            return pltpu.async_copy(
                table_hbm.at[idx_v.at[c]], buf_v.at[c % nbuf], gsems[c % nbuf]
            )

        def store_start(c):
            return pltpu.async_copy(
                buf_v.at[c % nbuf], out_hbm.at[pl.ds(base + c * chunk, chunk)],
                ssems[c % nbuf],
            )

        gh = [None] * n_chunks
        sh = [None] * n_chunks
        for c in range(nbuf - 1):
            gh[c] = gather_start(c)
        for c in range(n_chunks):
            if c + nbuf - 1 < n_chunks:
                if c >= 1:
                    sh[c - 1].wait()
                gh[c + nbuf - 1] = gather_start(c + nbuf - 1)
            gh[c].wait()
            sh[c] = store_start(c)
        for c in range(max(0, n_chunks - nbuf), n_chunks):
            sh[c].wait()

    return gather_kernel


def kernel(x, table):
    n = x.shape[0]
    d = table.shape[1]
    assert n % NW == 0, n
    rows_per_worker = n // NW
    chunk = _chunk_rows(rows_per_worker)
    xi = x.astype(jnp.int32).reshape(NW, rows_per_worker // chunk, chunk)
    return _build(rows_per_worker, chunk, d, table.dtype)(xi, table)


# speedup vs baseline: 2.0414x; 1.0043x over previous
import functools

import jax
import jax.numpy as jnp
from jax import lax
from jax.experimental import pallas as pl
from jax.experimental.pallas import tpu as pltpu
from jax.experimental.pallas import tpu_sc as plsc

NC = 2
NS = 16
NW = NC * NS
NBUF = 4


def _chunk_rows(rows_per_worker: int) -> int:
    best = 1
    for c in range(1, 257):
        if rows_per_worker % c == 0:
            best = c
    return best


@functools.cache
def _build(rows_per_worker: int, chunk: int, d: int, dtype):
    n_chunks = rows_per_worker // chunk
    nbuf = min(NBUF, n_chunks)
    mesh = plsc.VectorSubcoreMesh(core_axis_name="c", subcore_axis_name="s")

    @functools.partial(
        pl.kernel,
        mesh=mesh,
        out_type=jax.ShapeDtypeStruct((NW * rows_per_worker, d), dtype),
        scratch_types=[
            pltpu.VMEM((n_chunks, chunk), jnp.int32),
            pltpu.VMEM((nbuf, chunk, d), dtype),
            [pltpu.SemaphoreType.DMA] * nbuf,
            [pltpu.SemaphoreType.DMA] * nbuf,
        ],
        compiler_params=pltpu.CompilerParams(
            use_tc_tiling_on_sc=False,
            disable_bounds_checks=True,
            disable_semaphore_checks=True,
            skip_device_barrier=True,
        ),
    )
    def gather_kernel(idx_hbm, table_hbm, out_hbm, idx_v, buf_v, gsems, ssems):
        wid = lax.axis_index("s") * NC + lax.axis_index("c")
        base = wid * rows_per_worker
        pltpu.sync_copy(idx_hbm.at[wid], idx_v)

        def gather_start(c):
            return pltpu.async_copy(
                table_hbm.at[idx_v.at[c]], buf_v.at[c % nbuf], gsems[c % nbuf]
            )

        def store_start(c):
            return pltpu.async_copy(
                buf_v.at[c % nbuf], out_hbm.at[pl.ds(base + c * chunk, chunk)],
                ssems[c % nbuf],
            )

        gh = [None] * n_chunks
        sh = [None] * n_chunks
        for c in range(nbuf - 1):
            gh[c] = gather_start(c)
        for c in range(n_chunks):
            if c + nbuf - 1 < n_chunks:
                if c >= 1:
                    sh[c - 1].wait()
                gh[c + nbuf - 1] = gather_start(c + nbuf - 1)
            gh[c].wait()
            sh[c] = store_start(c)
        for c in range(max(0, n_chunks - nbuf), n_chunks):
            sh[c].wait()

    return gather_kernel


def kernel(x, table):
    n = x.shape[0]
    d = table.shape[1]
    assert n % NW == 0, n
    rows_per_worker = n // NW
    chunk = _chunk_rows(rows_per_worker)
    xi = x.astype(jnp.int32).reshape(NW, rows_per_worker // chunk, chunk)
    return _build(rows_per_worker, chunk, d, table.dtype)(xi, table)


# speedup vs baseline: 2.0556x; 1.0070x over previous
import functools

import jax
import jax.numpy as jnp
from jax import lax
from jax.experimental import pallas as pl
from jax.experimental.pallas import tpu as pltpu
from jax.experimental.pallas import tpu_sc as plsc

NC = 2
NS = 16
NW = NC * NS
NBUF = 6


def _chunk_rows(rows_per_worker: int) -> int:
    best = 1
    for c in range(1, 257):
        if rows_per_worker % c == 0:
            best = c
    return best


@functools.cache
def _build(rows_per_worker: int, chunk: int, d: int, dtype):
    n_chunks = rows_per_worker // chunk
    nbuf = min(NBUF, n_chunks)
    mesh = plsc.VectorSubcoreMesh(core_axis_name="c", subcore_axis_name="s")

    @functools.partial(
        pl.kernel,
        mesh=mesh,
        out_type=jax.ShapeDtypeStruct((NW * rows_per_worker, d), dtype),
        scratch_types=[
            pltpu.VMEM((n_chunks, chunk), jnp.int32),
            pltpu.VMEM((nbuf, chunk, d), dtype),
            [pltpu.SemaphoreType.DMA] * nbuf,
            [pltpu.SemaphoreType.DMA] * nbuf,
        ],
        compiler_params=pltpu.CompilerParams(use_tc_tiling_on_sc=False),
    )
    def gather_kernel(idx_hbm, table_hbm, out_hbm, idx_v, buf_v, gsems, ssems):
        wid = lax.axis_index("s") * NC + lax.axis_index("c")
        base = wid * rows_per_worker
        head = min(nbuf - 1, n_chunks)
        pltpu.sync_copy(
            idx_hbm.at[wid, pl.ds(0, head)], idx_v.at[pl.ds(0, head)]
        )

        def gather_start(c):
            return pltpu.async_copy(
                table_hbm.at[idx_v.at[c]], buf_v.at[c % nbuf], gsems[c % nbuf]
            )

        def store_start(c):
            return pltpu.async_copy(
                buf_v.at[c % nbuf], out_hbm.at[pl.ds(base + c * chunk, chunk)],
                ssems[c % nbuf],
            )

        gh = [None] * n_chunks
        sh = [None] * n_chunks
        for c in range(head):
            gh[c] = gather_start(c)
        if head < n_chunks:
            pltpu.sync_copy(
                idx_hbm.at[wid, pl.ds(head, n_chunks - head)],
                idx_v.at[pl.ds(head, n_chunks - head)],
            )
        for c in range(n_chunks):
            if c + nbuf - 1 < n_chunks:
                if c >= 1:
                    sh[c - 1].wait()
                gh[c + nbuf - 1] = gather_start(c + nbuf - 1)
            gh[c].wait()
            sh[c] = store_start(c)
        for c in range(max(0, n_chunks - nbuf), n_chunks):
            sh[c].wait()

    return gather_kernel


def kernel(x, table):
    n = x.shape[0]
    d = table.shape[1]
    assert n % NW == 0, n
    rows_per_worker = n // NW
    chunk = _chunk_rows(rows_per_worker)
    xi = x.astype(jnp.int32).reshape(NW, rows_per_worker // chunk, chunk)
    return _build(rows_per_worker, chunk, d, table.dtype)(xi, table)
